# baseline (speedup 1.0000x reference)
import functools

import jax
import jax.numpy as jnp
from jax import lax
from jax.experimental import pallas as pl
from jax.experimental.pallas import tpu as pltpu
from jax.experimental.pallas import tpu_sc as plsc

E = 64
B = 4096
S = 200
NC = 2
NSUB = 16
NW = NC * NSUB
BBLK = B // NW
NS = 4
NCHUNK = S // NS
L = 16


@jax.jit
def _run(xt, table, pos):
    mesh = plsc.VectorSubcoreMesh(core_axis_name="c", subcore_axis_name="s")

    @functools.partial(
        pl.kernel,
        mesh=mesh,
        compiler_params=pltpu.CompilerParams(
            use_tc_tiling_on_sc=False, needs_layout_passes=False),
        out_type=jax.ShapeDtypeStruct((S, E // 8, NW, 8, BBLK), jnp.float32),
        scratch_types=[
            pltpu.VMEM((2, NS, BBLK), jnp.int32),
            pltpu.VMEM((2, NS, BBLK, E), jnp.float32),
            pltpu.VMEM((NS, E // 8, 8, BBLK), jnp.float32),
            pltpu.VMEM((S, E), jnp.float32),
            pltpu.SemaphoreType.DMA,
            pltpu.SemaphoreType.DMA,
        ],
    )
    def body(xt_hbm, table_hbm, pos_hbm, out_hbm,
             idx_v, rows_v, stage_v, pos_v, sem0, sem1):
        wid = lax.axis_index("s") * NC + lax.axis_index("c")
        sems = (sem0, sem1)
        pltpu.sync_copy(pos_hbm, pos_v)

        def fetch(c, buf):
            pltpu.sync_copy(
                xt_hbm.at[pl.ds(c * NS, NS), pl.ds(wid * BBLK, BBLK)],
                idx_v.at[buf])
            for j in range(NS):
                pltpu.async_copy(
                    table_hbm.at[idx_v.at[buf, j]], rows_v.at[buf, j],
                    sems[buf])

        def drain(buf):
            for j in range(NS):
                pltpu.make_async_copy(
                    table_hbm.at[idx_v.at[buf, j]], rows_v.at[buf, j],
                    sems[buf]).wait()

        fetch(0, 0)

        def chunk_pair(c, carry):
            for b in range(2):
                cc = c + b

                @pl.when(cc + 1 < NCHUNK)
                def _():
                    fetch(cc + 1, 1 - b)

                drain(b)

                def trans_e(i, carry2):
                    j = i // (E // L)
                    k = i % (E // L)
                    pvec = pos_v[cc * NS + j, pl.ds(k * L, L)]
                    for e8i in range(L):
                        e = k * L + e8i
                        et = e // 8
                        p = pvec[e8i]
                        for q in range(BBLK // L):
                            col = plsc.load_gather(
                                rows_v,
                                [jnp.full((L,), b, jnp.int32),
                                 jnp.full((L,), j, jnp.int32),
                                 lax.iota(jnp.int32, L) + q * L,
                                 jnp.full((L,), e, jnp.int32)])
                            stage_v[j, et, e % 8, pl.ds(q * L, L)] = col + p
                    return carry2

                lax.fori_loop(0, NS * (E // L), trans_e, 0)
                pltpu.sync_copy(
                    stage_v,
                    out_hbm.at[pl.ds(cc * NS, NS), :, wid])
            return carry

        lax.fori_loop(0, NCHUNK // 2, lambda i, c: chunk_pair(i * 2, c), 0)

    return body(xt, table, pos)


def kernel(x, table, pos_encoding):
    xt = x.T
    pos = pos_encoding[:S]
    a = _run(xt, table, pos)
    return a.transpose(2, 4, 0, 1, 3).reshape(B, S, E)


# speedup vs baseline: 1.1167x; 1.1167x over previous
import functools

import jax
import jax.numpy as jnp
from jax import lax
from jax.experimental import pallas as pl
from jax.experimental.pallas import tpu as pltpu
from jax.experimental.pallas import tpu_sc as plsc

E = 64
B = 4096
S = 200
NC = 2
NSUB = 16
NW = NC * NSUB
BBLK = B // NW
NS = 4
NCHUNK = S // NS
L = 16

@jax.jit
def _run(xt, table, pos):
    mesh = plsc.VectorSubcoreMesh(core_axis_name="c", subcore_axis_name="s")

    @functools.partial(
        pl.kernel,
        mesh=mesh,
        compiler_params=pltpu.CompilerParams(
            use_tc_tiling_on_sc=False, needs_layout_passes=False),
        out_type=jax.ShapeDtypeStruct((S, E // 8, NW, 8, BBLK), jnp.float32),
        scratch_types=[
            pltpu.VMEM((2, NS, BBLK), jnp.int32),
            pltpu.VMEM((NS * BBLK, E), jnp.float32),
            pltpu.VMEM((NS * BBLK, E), jnp.float32),
            pltpu.VMEM((NS, E // 8, 8, BBLK), jnp.float32),
            pltpu.VMEM((S, E), jnp.float32),
            pltpu.SemaphoreType.DMA,
            pltpu.SemaphoreType.DMA,
        ],
    )
    def body(xt_hbm, table_hbm, pos_hbm, out_hbm,
             idx_v, rows0_v, rows1_v, stage_v, pos_v, sem0, sem1):
        wid = lax.axis_index("s") * NC + lax.axis_index("c")
        rows_bufs = (rows0_v, rows1_v)
        sems = (sem0, sem1)
        pltpu.sync_copy(pos_hbm, pos_v)

        def fetch(c, buf):
            pltpu.sync_copy(
                xt_hbm.at[pl.ds(c * NS, NS), pl.ds(wid * BBLK, BBLK)],
                idx_v.at[buf])
            for j in range(NS):
                pltpu.async_copy(
                    table_hbm.at[idx_v.at[buf, j]],
                    rows_bufs[buf].at[pl.ds(j * BBLK, BBLK)],
                    sems[buf])

        def drain(buf):
            for j in range(NS):
                pltpu.make_async_copy(
                    table_hbm.at[idx_v.at[buf, j]],
                    rows_bufs[buf].at[pl.ds(j * BBLK, BBLK)],
                    sems[buf]).wait()

        fetch(0, 0)

        lanes = lax.iota(jnp.int32, L)
        etv = [(lanes + k * L) // 8 for k in range(E // L)]
        e8v = [(lanes + k * L) % 8 for k in range(E // L)]

        def chunk_pair(c, carry):
            for b in range(2):
                cc = c + b

                @pl.when(cc + 1 < NCHUNK)
                def _():
                    fetch(cc + 1, 1 - b)

                drain(b)
                rows = rows_bufs[b]

                for j in range(NS):
                    jv = jnp.full((L,), j, jnp.int32)
                    pv = [pos_v[cc * NS + j, pl.ds(k * L, L)]
                          for k in range(E // L)]

                    def b_body(bb, carry2, j=j, jv=jv, pv=pv):
                        bv = jnp.full((L,), bb, jnp.int32)
                        for k in range(E // L):
                            vec = rows[j * BBLK + bb, pl.ds(k * L, L)] + pv[k]
                            plsc.store_scatter(
                                stage_v, [jv, etv[k], e8v[k], bv], vec)
                        return carry2

                    lax.fori_loop(0, BBLK, b_body, 0)

                pltpu.sync_copy(
                    stage_v,
                    out_hbm.at[pl.ds(cc * NS, NS), :, wid])
            return carry

        lax.fori_loop(0, NCHUNK // 2, lambda i, c: chunk_pair(i * 2, c), 0)

    return body(xt, table, pos)


def kernel(x, table, pos_encoding):
    xt = x.T
    pos = pos_encoding[:S]
    a = _run(xt, table, pos)
    return a.transpose(2, 4, 0, 1, 3).reshape(B, S, E)


# speedup vs baseline: 1.1274x; 1.0097x over previous
import functools

import jax
import jax.numpy as jnp
from jax import lax
from jax.experimental import pallas as pl
from jax.experimental.pallas import tpu as pltpu
from jax.experimental.pallas import tpu_sc as plsc

E = 64
B = 4096
S = 200
NC = 2
NSUB = 16
NW = NC * NSUB
BBLK = B // NW
NS = 4
NCHUNK = S // NS
L = 16

@jax.jit
def _run(xt, table, pos):
    mesh = plsc.VectorSubcoreMesh(core_axis_name="c", subcore_axis_name="s")

    @functools.partial(
        pl.kernel,
        mesh=mesh,
        compiler_params=pltpu.CompilerParams(
            use_tc_tiling_on_sc=False, needs_layout_passes=False),
        out_type=jax.ShapeDtypeStruct((S, E // 8, NW, 8, BBLK), jnp.float32),
        scratch_types=[
            pltpu.VMEM((2, NS, BBLK), jnp.int32),
            pltpu.VMEM((NS * BBLK, E), jnp.float32),
            pltpu.VMEM((NS * BBLK, E), jnp.float32),
            pltpu.VMEM((NS, E // 8, 8, BBLK), jnp.float32),
            pltpu.VMEM((S, E), jnp.float32),
            pltpu.SemaphoreType.DMA,
            pltpu.SemaphoreType.DMA,
        ],
    )
    def body(xt_hbm, table_hbm, pos_hbm, out_hbm,
             idx_v, rows0_v, rows1_v, stage_v, pos_v, sem0, sem1):
        wid = lax.axis_index("s") * NC + lax.axis_index("c")
        rows_bufs = (rows0_v, rows1_v)
        sems = (sem0, sem1)
        pltpu.sync_copy(pos_hbm, pos_v)

        def fetch(c, buf):
            pltpu.sync_copy(
                xt_hbm.at[pl.ds(c * NS, NS), pl.ds(wid * BBLK, BBLK)],
                idx_v.at[buf])
            for j in range(NS):
                pltpu.async_copy(
                    table_hbm.at[idx_v.at[buf, j]],
                    rows_bufs[buf].at[pl.ds(j * BBLK, BBLK)],
                    sems[buf])

        def drain(buf):
            for j in range(NS):
                pltpu.make_async_copy(
                    table_hbm.at[idx_v.at[buf, j]],
                    rows_bufs[buf].at[pl.ds(j * BBLK, BBLK)],
                    sems[buf]).wait()

        fetch(0, 0)

        lanes = lax.iota(jnp.int32, L)
        etv = [(lanes + k * L) // 8 for k in range(E // L)]
        e8v = [(lanes + k * L) % 8 for k in range(E // L)]

        def chunk_pair(c, carry):
            for b in range(2):
                cc = c + b

                @pl.when(cc + 1 < NCHUNK)
                def _():
                    fetch(cc + 1, 1 - b)

                drain(b)
                rows = rows_bufs[b]

                for j in range(NS):
                    jv = jnp.full((L,), j, jnp.int32)
                    pv = [pos_v[cc * NS + j, pl.ds(k * L, L)]
                          for k in range(E // L)]

                    def b_body(i, carry2, j=j, jv=jv, pv=pv):
                        for bu in range(8):
                            bb = i * 8 + bu
                            bv = jnp.full((L,), bb, jnp.int32)
                            for k in range(E // L):
                                vec = (rows[j * BBLK + bb, pl.ds(k * L, L)]
                                       + pv[k])
                                plsc.store_scatter(
                                    stage_v, [jv, etv[k], e8v[k], bv], vec)
                        return carry2

                    lax.fori_loop(0, BBLK // 8, b_body, 0)

                pltpu.sync_copy(
                    stage_v,
                    out_hbm.at[pl.ds(cc * NS, NS), :, wid])
            return carry

        lax.fori_loop(0, NCHUNK // 2, lambda i, c: chunk_pair(i * 2, c), 0)

    return body(xt, table, pos)


def kernel(x, table, pos_encoding):
    xt = x.T
    pos = pos_encoding[:S]
    a = _run(xt, table, pos)
    return a.transpose(2, 4, 0, 1, 3).reshape(B, S, E)
